# speedup vs baseline: 1.0996x; 1.0996x over previous
import functools

import jax
import jax.numpy as jnp
from jax.experimental import pallas as pl
from jax.experimental.pallas import tpu as pltpu

HIDDEN = 2048
NUM_EXPERTS = 8
TOP_K = 2
F_TEXT = 1024
SHARED_F = 2048
NORM_MIN = 1e-12
T = 2048
BM = 256
NT = T // BM


def _router_body(x_ref, wt_ref, bias_ref, logits_ref, combine_ref):
    x = x_ref[...]
    wt = wt_ref[...]
    logits = jnp.dot(x, wt, preferred_element_type=jnp.float32)
    logits_ref[...] = logits
    m = jnp.max(logits, axis=1, keepdims=True)
    ex = jnp.exp(logits - m)
    probs = ex / jnp.sum(ex, axis=1, keepdims=True)
    corrected = probs + bias_ref[...]
    idx = jax.lax.broadcasted_iota(jnp.int32, corrected.shape, 1)
    neg_inf = jnp.float32(-jnp.inf)
    big = jnp.int32(NUM_EXPERTS)

    def top1(c):
        m1 = jnp.max(c, axis=1, keepdims=True)
        is1 = c == m1
        a1 = jnp.min(jnp.where(is1, idx, big), axis=1, keepdims=True)
        return a1

    a1 = top1(corrected)
    c2 = jnp.where(idx == a1, neg_inf, corrected)
    a2 = top1(c2)
    p1 = jnp.sum(jnp.where(idx == a1, probs, 0.0), axis=1, keepdims=True)
    p2 = jnp.sum(jnp.where(idx == a2, probs, 0.0), axis=1, keepdims=True)
    denom = jnp.maximum(p1 + p2, NORM_MIN)
    w1 = p1 / denom
    w2 = p2 / denom
    combine_ref[...] = jnp.where(idx == a1, w1, 0.0) + jnp.where(idx == a2, w2, 0.0)


def _experts_body(x_ref, gu_ref, dn_ref, comb_ref, out_ref):
    e = pl.program_id(0)
    t = pl.program_id(1)
    xt = x_ref[pl.ds(t * BM, BM), :]
    gu = jnp.dot(xt, gu_ref[0], preferred_element_type=jnp.float32)
    g = gu[:, :F_TEXT]
    u = gu[:, F_TEXT:]
    h = (g * jax.nn.sigmoid(g)) * u
    out_e = jnp.dot(h.astype(jnp.bfloat16), dn_ref[0],
                    preferred_element_type=jnp.float32)
    comb = comb_ref[pl.ds(t * BM, BM), :]
    eidx = jax.lax.broadcasted_iota(jnp.int32, comb.shape, 1)
    w = jnp.sum(jnp.where(eidx == e, comb, 0.0), axis=1, keepdims=True)
    contrib = out_e * w

    @pl.when(e == 0)
    def _():
        out_ref[pl.ds(t * BM, BM), :] = contrib

    @pl.when(e != 0)
    def _():
        out_ref[pl.ds(t * BM, BM), :] = out_ref[pl.ds(t * BM, BM), :] + contrib


def _shared_body(x_ref, gw_ref, uw_ref, dw_ref, moe_ref, out_ref):
    xt = x_ref[...]
    g = jnp.dot(xt, gw_ref[...], preferred_element_type=jnp.float32)
    u = jnp.dot(xt, uw_ref[...], preferred_element_type=jnp.float32)
    h = (g * jax.nn.sigmoid(g)) * u
    out = jnp.dot(h.astype(jnp.bfloat16), dw_ref[...],
                  preferred_element_type=jnp.float32)
    out_ref[...] = out + moe_ref[...]


@jax.jit
def _run(x, router_weight, e_score_bias, gate_up_proj, down_proj,
         shared_gate_w, shared_up_w, shared_down_w):
    xf = x.reshape(T, HIDDEN)
    logits, combine = pl.pallas_call(
        _router_body,
        grid=(1,),
        in_specs=[
            pl.BlockSpec((T, HIDDEN), lambda i: (0, 0)),
            pl.BlockSpec((HIDDEN, NUM_EXPERTS), lambda i: (0, 0)),
            pl.BlockSpec((1, NUM_EXPERTS), lambda i: (0, 0)),
        ],
        out_specs=[
            pl.BlockSpec((T, NUM_EXPERTS), lambda i: (0, 0)),
            pl.BlockSpec((T, NUM_EXPERTS), lambda i: (0, 0)),
        ],
        out_shape=[
            jax.ShapeDtypeStruct((T, NUM_EXPERTS), jnp.float32),
            jax.ShapeDtypeStruct((T, NUM_EXPERTS), jnp.float32),
        ],
    )(xf, router_weight.T, e_score_bias)

    xb = xf.astype(jnp.bfloat16)
    gub = gate_up_proj.astype(jnp.bfloat16)
    dnb = down_proj.astype(jnp.bfloat16)

    moe_out = pl.pallas_call(
        _experts_body,
        grid=(NUM_EXPERTS, NT),
        in_specs=[
            pl.BlockSpec((T, HIDDEN), lambda e, t: (0, 0)),
            pl.BlockSpec((1, HIDDEN, 2 * F_TEXT), lambda e, t: (e, 0, 0)),
            pl.BlockSpec((1, F_TEXT, HIDDEN), lambda e, t: (e, 0, 0)),
            pl.BlockSpec((T, NUM_EXPERTS), lambda e, t: (0, 0)),
        ],
        out_specs=pl.BlockSpec((T, HIDDEN), lambda e, t: (0, 0)),
        out_shape=jax.ShapeDtypeStruct((T, HIDDEN), jnp.float32),
        compiler_params=pltpu.CompilerParams(
            dimension_semantics=("arbitrary", "arbitrary"),
        ),
    )(xb, gub, dnb, combine)

    gwb = shared_gate_w.T.astype(jnp.bfloat16)
    uwb = shared_up_w.T.astype(jnp.bfloat16)
    dwb = shared_down_w.T.astype(jnp.bfloat16)
    final = pl.pallas_call(
        _shared_body,
        grid=(NT,),
        in_specs=[
            pl.BlockSpec((BM, HIDDEN), lambda t: (t, 0)),
            pl.BlockSpec((HIDDEN, SHARED_F), lambda t: (0, 0)),
            pl.BlockSpec((HIDDEN, SHARED_F), lambda t: (0, 0)),
            pl.BlockSpec((SHARED_F, HIDDEN), lambda t: (0, 0)),
            pl.BlockSpec((BM, HIDDEN), lambda t: (t, 0)),
        ],
        out_specs=pl.BlockSpec((BM, HIDDEN), lambda t: (t, 0)),
        out_shape=jax.ShapeDtypeStruct((T, HIDDEN), jnp.float32),
    )(xb, gwb, uwb, dwb, moe_out)

    return final.reshape(1, T, HIDDEN), logits


def kernel(hidden_states, router_weight, e_score_bias, gate_up_proj, down_proj,
           shared_gate_w, shared_up_w, shared_down_w):
    return _run(hidden_states, router_weight, e_score_bias, gate_up_proj,
                down_proj, shared_gate_w, shared_up_w, shared_down_w)
